# baseline (speedup 1.0000x reference)
import functools

import jax
import jax.numpy as jnp
from jax import lax
from jax.experimental import pallas as pl
from jax.experimental.pallas import tpu as pltpu
from jax.experimental.pallas import tpu_sc as plsc

N_USERS = 25000
N_ITEMS = 25000
N = N_USERS + N_ITEMS
D = 64
H = 32
NL = 3
EPS = 0.02
CL_W = 0.001
CL_T = 0.2
DECAY = 1e-4
E = 800000
B = 4096

SUB = 128
SUBS_PER_TILE = 408
E2 = SUBS_PER_TILE * 16 * SUB
EROWS = E2 // SUB
IBLK = 6
NBLK = SUBS_PER_TILE // IBLK
NT = IBLK // 3
FCH = 125
FITERS = (N // FCH) // 16

_MESH = plsc.VectorSubcoreMesh(core_axis_name="c", subcore_axis_name="s")


def _prop_body(with_noise, *refs):
    if with_noise:
        (x0_r, ed_r, unit3_r,
         xa_r, xb_r, aca_r, acb_r,
         edata, rows, fb, acc_sp,
         semi0, semi1, semg0, semg1, semg2) = refs
    else:
        (x0_r, ed_r,
         xa_r, xb_r, aca_r, acb_r,
         edata, rows, fb, acc_sp,
         semi0, semi1, semg0, semg1, semg2) = refs
        unit3_r = None

    c = lax.axis_index("c")
    s = lax.axis_index("s")
    semi = (semi0, semi1)
    semg = (semg0, semg1, semg2)

    seg_b = (rows.at[0].at[pl.ds(0, FCH)], rows.at[1].at[pl.ds(0, FCH)])
    aux_b = (rows.at[2].at[pl.ds(0, FCH)], fb.at[0])
    acm_b = (fb.at[1], fb.at[2])
    z16 = jnp.zeros((16,), jnp.float32)
    sub0 = s * SUBS_PER_TILE

    def _one_layer(x_r, xo_r, acci_r, acco_r, unit_r):
        def _zbody(i, _):
            rows[0, i, 0:16] = z16
            rows[0, i, 16:32] = z16
            return 0

        lax.fori_loop(0, FCH, _zbody, 0)

        def _zcopy(k, _):
            pltpu.sync_copy(seg_b[0],
                            acc_sp.at[pl.ds((s * FITERS + k) * FCH, FCH)])
            return 0

        lax.fori_loop(0, FITERS, _zcopy, 0)
        plsc.subcore_barrier()

        def _load_blk(bb, bslot):
            pltpu.async_copy(ed_r.at[pl.ds(sub0 + bb * IBLK, IBLK)],
                             edata.at[bslot], semi[bslot])

        def _drain_blk(bslot):
            pltpu.make_async_copy(ed_r.at[pl.ds(0, IBLK)], edata.at[bslot],
                                  semi[bslot]).wait()

        def _issue_gather(bslot, jj, u):
            pltpu.async_copy(x_r.at[c].at[edata.at[bslot].at[jj].at[0]],
                             rows.at[u], semg[u])

        def _drain_gather(u):
            pltpu.make_async_copy(x_r.at[c].at[pl.ds(0, SUB)], rows.at[u],
                                  semg[u]).wait()

        def _scat_sync(bslot, jj, u):
            pltpu.sync_copy(rows.at[u],
                            acc_sp.at[edata.at[bslot].at[jj].at[1]],
                            add=True)

        def _scale(bslot, jj, u):
            def _sc16(q, _):
                wbits = edata[bslot, jj, 2, pl.ds(q * 16, 16)]
                w16 = lax.bitcast_convert_type(wbits, jnp.float32)
                for t in range(16):
                    e16 = q * 16 + t
                    wv = w16[t]
                    rows[u, e16, 0:16] = rows[u, e16, 0:16] * wv
                    rows[u, e16, 16:32] = rows[u, e16, 16:32] * wv
                return 0

            lax.fori_loop(0, SUB // 16, _sc16, 0)

        def _sub_step(bb, bslot, nbslot, t, u):
            jj = 3 * t + u
            un = (u + 1) % 3

            if u < 2:
                _issue_gather(bslot, jj + 1, un)
            else:
                @pl.when(t == 0)
                def _():
                    _issue_gather(bslot, 3, un)

                    @pl.when(bb + 1 < NBLK)
                    def _():
                        _load_blk(bb + 1, nbslot)

                @pl.when(t == NT - 1)
                def _():
                    @pl.when(bb + 1 < NBLK)
                    def _():
                        _drain_blk(nbslot)
                        _issue_gather(nbslot, 0, un)

            _drain_gather(u)
            _scale(bslot, jj, u)
            _scat_sync(bslot, jj, u)

        _load_blk(0, 0)
        _drain_blk(0)
        _issue_gather(0, 0, 0)

        def _bpair(bp, _):
            def _triad(t, _2):
                for u in range(3):
                    _sub_step(2 * bp, 0, 1, t, u)
                return 0

            lax.fori_loop(0, NT, _triad, 0)

            def _triad2(t, _2):
                for u in range(3):
                    _sub_step(2 * bp + 1, 1, 0, t, u)
                return 0

            lax.fori_loop(0, NT, _triad2, 0)
            return 0

        lax.fori_loop(0, NBLK // 2, _bpair, 0)
        plsc.subcore_barrier()

        def _flush(k, _):
            r0 = (s * FITERS + k) * FCH
            pltpu.sync_copy(acc_sp.at[pl.ds(r0, FCH)], seg_b[0])
            if with_noise:
                pltpu.sync_copy(unit_r.at[c].at[pl.ds(r0, FCH)], aux_b[0])
            pltpu.sync_copy(acci_r.at[c].at[pl.ds(r0, FCH)], acm_b[0])

            def _vb(i, _):
                s0 = rows[0, i, 0:16]
                s1 = rows[0, i, 16:32]
                if with_noise:
                    s0 = s0 + rows[2, i, 0:16] * jnp.sign(s0)
                    s1 = s1 + rows[2, i, 16:32] * jnp.sign(s1)
                    rows[0, i, 0:16] = s0
                    rows[0, i, 16:32] = s1
                fb[1, i, 0:16] = fb[1, i, 0:16] + s0
                fb[1, i, 16:32] = fb[1, i, 16:32] + s1
                return 0

            lax.fori_loop(0, FCH, _vb, 0)
            pltpu.sync_copy(seg_b[0], xo_r.at[c].at[pl.ds(r0, FCH)])
            pltpu.sync_copy(acm_b[0], acco_r.at[c].at[pl.ds(r0, FCH)])
            return 0

        lax.fori_loop(0, FITERS, _flush, 0)
        plsc.subcore_barrier()

    u0 = unit3_r.at[0] if with_noise else None
    u1 = unit3_r.at[1] if with_noise else None
    u2 = unit3_r.at[2] if with_noise else None
    _one_layer(x0_r, xa_r, x0_r, aca_r, u0)
    _one_layer(xa_r, xb_r, aca_r, acb_r, u1)
    _one_layer(xb_r, xa_r, acb_r, aca_r, u2)


def _make_prop(with_noise):
    out_type = tuple(
        jax.ShapeDtypeStruct((2, N, H), jnp.float32) for _ in range(4)
    )
    scratch = [
        pltpu.VMEM((2, IBLK, 3, SUB), jnp.int32),
        pltpu.VMEM((3, SUB, H), jnp.float32),
        pltpu.VMEM((3, FCH, H), jnp.float32),
        pltpu.VMEM_SHARED((N, H), jnp.float32),
    ] + [pltpu.SemaphoreType.DMA] * 5
    return pl.kernel(
        functools.partial(_prop_body, with_noise),
        out_type=out_type,
        mesh=_MESH,
        scratch_types=scratch,
        compiler_params=pltpu.CompilerParams(use_tc_tiling_on_sc=False),
    )


_prop_plain = _make_prop(False)
_prop_noise = _make_prop(True)


def _gather_body(m0, a1, a2, uemb, iemb, uix, pix, nix, pixn, nixn,
                 u_o, pi_o, ni_o, z1u_o, z2u_o, z1i_o, z2i_o,
                 u0_o, pi0_o, ni0_o,
                 idx_v, rows_v, idx1_v, rows64_v, sem):
    c = lax.axis_index("c")
    s = lax.axis_index("s")
    wid = s * 2 + c

    def split_gather(idx_hbm, table, out, load_idx):
        if load_idx:
            pltpu.sync_copy(idx_hbm.at[pl.ds(s * 256, 256)], idx_v)
        for j in range(2):
            pltpu.async_copy(table.at[c].at[idx_v.at[pl.ds(j * 128, 128)]],
                             rows_v.at[j], sem).wait()
            pltpu.sync_copy(rows_v.at[j],
                            out.at[c].at[pl.ds(s * 256 + j * 128, 128)])

    split_gather(uix, m0, u_o, True)
    split_gather(uix, a1, z1u_o, False)
    split_gather(uix, a2, z2u_o, False)
    split_gather(pixn, m0, pi_o, True)
    split_gather(pixn, a1, z1i_o, False)
    split_gather(pixn, a2, z2i_o, False)
    split_gather(nixn, m0, ni_o, True)

    def raw_gather(idx_hbm, table, out):
        pltpu.sync_copy(idx_hbm.at[pl.ds(wid * 128, 128)], idx1_v)
        pltpu.async_copy(table.at[idx1_v], rows64_v, sem).wait()
        pltpu.sync_copy(rows64_v, out.at[pl.ds(wid * 128, 128)])

    raw_gather(uix, uemb, u0_o)
    raw_gather(pix, iemb, pi0_o)
    raw_gather(nix, iemb, ni0_o)


_gather_k = pl.kernel(
    _gather_body,
    out_type=tuple(
        [jax.ShapeDtypeStruct((2, B, H), jnp.float32)] * 7
        + [jax.ShapeDtypeStruct((B, D), jnp.float32)] * 3
    ),
    mesh=_MESH,
    scratch_types=[
        pltpu.VMEM((256,), jnp.int32),
        pltpu.VMEM((2, 128, H), jnp.float32),
        pltpu.VMEM((128,), jnp.int32),
        pltpu.VMEM((128, D), jnp.float32),
        pltpu.SemaphoreType.DMA,
    ],
    compiler_params=pltpu.CompilerParams(use_tc_tiling_on_sc=False),
)


PREP_R = 1000
PREP_STEPS = N // PREP_R


def _prep_body(ue, ie, r0, r1, r2, r3, r4, r5,
               x0_o, o0, o1, o2, o3, o4, o5):
    i = pl.program_id(0)
    emb = jnp.where(i < (N_USERS // PREP_R), ue[...], ie[...])
    x0_o[0] = emb[:, :H]
    x0_o[1] = emb[:, H:]
    for r, o in ((r0, o0), (r1, o1), (r2, o2), (r3, o3), (r4, o4), (r5, o5)):
        x = r[...]
        nrm = jnp.sqrt(jnp.sum(x * x, axis=1, keepdims=True))
        un = x / (nrm + 1e-12) * EPS
        o[0] = un[:, :H]
        o[1] = un[:, H:]


def _prep(user_emb, item_emb, rs):
    half = N_USERS // PREP_R
    r_spec = pl.BlockSpec((PREP_R, D), lambda i: (i, 0))
    o_spec = pl.BlockSpec((2, PREP_R, H), lambda i: (0, i, 0))
    return pl.pallas_call(
        _prep_body,
        grid=(PREP_STEPS,),
        in_specs=[
            pl.BlockSpec((PREP_R, D), lambda i: (jnp.minimum(i, half - 1), 0)),
            pl.BlockSpec((PREP_R, D), lambda i: (jnp.maximum(i - half, 0), 0)),
        ] + [r_spec] * 6,
        out_specs=[o_spec] * 7,
        out_shape=[jax.ShapeDtypeStruct((2, N, H), jnp.float32)] * 7,
    )(user_emb, item_emb, *rs)


RB = 512
RSTEPS = B // RB


def _nce_sum(z1b, z2f, z2b):
    z1lo = z1b[0] * 0.25
    z1hi = z1b[1] * 0.25
    n1 = jnp.sqrt(jnp.sum(z1lo * z1lo, axis=1, keepdims=True)
                  + jnp.sum(z1hi * z1hi, axis=1, keepdims=True))
    i1 = 1.0 / (n1 + 1e-12)
    z1lo = z1lo * i1
    z1hi = z1hi * i1
    z2lo = z2f[0] * 0.25
    z2hi = z2f[1] * 0.25
    n2 = jnp.sqrt(jnp.sum(z2lo * z2lo, axis=1, keepdims=True)
                  + jnp.sum(z2hi * z2hi, axis=1, keepdims=True))
    i2 = 1.0 / (n2 + 1e-12)
    z2lo = z2lo * i2
    z2hi = z2hi * i2
    nt = (((1,), (1,)), ((), ()))
    sim = (lax.dot_general(z1lo, z2lo, nt, preferred_element_type=jnp.float32)
           + lax.dot_general(z1hi, z2hi, nt, preferred_element_type=jnp.float32))
    sim = sim * (1.0 / CL_T)
    m = jnp.max(sim, axis=1, keepdims=True)
    logz = jnp.log(jnp.sum(jnp.exp(sim - m), axis=1, keepdims=True)) + m
    z2blo = z2b[0] * 0.25
    z2bhi = z2b[1] * 0.25
    nb = jnp.sqrt(jnp.sum(z2blo * z2blo, axis=1, keepdims=True)
                  + jnp.sum(z2bhi * z2bhi, axis=1, keepdims=True))
    ib = 1.0 / (nb + 1e-12)
    diag = (jnp.sum(z1lo * (z2blo * ib), axis=1, keepdims=True)
            + jnp.sum(z1hi * (z2bhi * ib), axis=1, keepdims=True)) * (1.0 / CL_T)
    return jnp.sum(logz - diag)


def _loss_body(z1u, z2uf, z2ub, z1i, z2if, z2ib, u, pi, ni, u0, pi0, ni0,
               loss_o, bpr_o, acc):
    i = pl.program_id(0)

    @pl.when(i == 0)
    def _():
        ps = jnp.sum(u[0] * pi[0] + u[1] * pi[1], axis=1) * (1.0 / 16.0)
        ns = jnp.sum(u[0] * ni[0] + u[1] * ni[1], axis=1) * (1.0 / 16.0)
        d = ps - ns
        sp = jnp.maximum(-d, 0.0) + jnp.log(1.0 + jnp.exp(-jnp.abs(d)))
        acc[0] = jnp.mean(sp)
        acc[1] = (jnp.sum(u0[...] * u0[...]) + jnp.sum(pi0[...] * pi0[...])
                  + jnp.sum(ni0[...] * ni0[...])) * (1.0 / B)
        acc[2] = 0.0
        acc[3] = 0.0

    acc[2] += _nce_sum(z1u, z2uf, z2ub)
    acc[3] += _nce_sum(z1i, z2if, z2ib)

    @pl.when(i == RSTEPS - 1)
    def _():
        cl = acc[2] * (1.0 / B) + acc[3] * (1.0 / B)
        loss_o[...] = jnp.broadcast_to(acc[0] + DECAY * acc[1] + CL_W * cl,
                                       (1, 1))
        bpr_o[...] = jnp.broadcast_to(acc[0], (1, 1))


def _loss(z1u, z2u, z1i, z2i, u, pi, ni, u0, pi0, ni0):
    full = pl.BlockSpec((2, B, H), lambda i: (0, 0, 0))
    blk = pl.BlockSpec((2, RB, H), lambda i: (0, i, 0))
    raw = pl.BlockSpec((B, D), lambda i: (0, 0))
    return pl.pallas_call(
        _loss_body,
        grid=(RSTEPS,),
        in_specs=[blk, full, blk, blk, full, blk,
                  full, full, full, raw, raw, raw],
        out_specs=[pl.BlockSpec((1, 1), lambda i: (0, 0))] * 2,
        out_shape=[jax.ShapeDtypeStruct((1, 1), jnp.float32)] * 2,
        scratch_shapes=[pltpu.SMEM((4,), jnp.float32)],
    )(z1u, z2u, z2u, z1i, z2i, z2i, u, pi, ni, u0, pi0, ni0)


def kernel(users, pos_items, neg_items, edge_index, edge_weight,
           user_emb, item_emb):
    users = users.astype(jnp.int32)
    pos_items = pos_items.astype(jnp.int32)
    neg_items = neg_items.astype(jnp.int32)
    pad = E2 - E
    pad_idx = (jnp.arange(pad, dtype=jnp.int32) * 37) % N
    src = jnp.concatenate(
        [edge_index[0].astype(jnp.int32), pad_idx]).reshape(EROWS, 1, SUB)
    dst = jnp.concatenate(
        [edge_index[1].astype(jnp.int32), pad_idx]).reshape(EROWS, 1, SUB)
    wbits = lax.bitcast_convert_type(
        jnp.concatenate([edge_weight, jnp.zeros((pad,), jnp.float32)]),
        jnp.int32).reshape(EROWS, 1, SUB)
    ed = jnp.concatenate([src, dst, wbits], axis=1)

    rs = [jax.random.uniform(jax.random.fold_in(jax.random.key(k), l),
                             (N, D), dtype=jnp.float32)
          for k in (1, 2) for l in range(NL)]
    x0, u1a, u1b, u1c, u2a, u2b, u2c = _prep(user_emb, item_emb, rs)

    m0 = _prop_plain(x0, ed)[2]
    a1 = _prop_noise(x0, ed, jnp.stack([u1a, u1b, u1c]))[2]
    a2 = _prop_noise(x0, ed, jnp.stack([u2a, u2b, u2c]))[2]

    uix = users
    pix = pos_items
    nix = neg_items
    pixn = pix + N_USERS
    nixn = nix + N_USERS

    (u, pi, ni, z1u, z2u, z1i, z2i, u0, pi0, ni0) = _gather_k(
        m0, a1, a2, user_emb, item_emb, uix, pix, nix, pixn, nixn)

    loss_a, bpr_a = _loss(z1u, z2u, z1i, z2i, u, pi, ni, u0, pi0, ni0)
    return (loss_a[0, 0], bpr_a[0, 0])


# speedup vs baseline: 1.1268x; 1.1268x over previous
import functools

import jax
import jax.numpy as jnp
from jax import lax
from jax.experimental import pallas as pl
from jax.experimental.pallas import tpu as pltpu
from jax.experimental.pallas import tpu_sc as plsc

N_USERS = 25000
N_ITEMS = 25000
N = N_USERS + N_ITEMS
D = 64
H = 32
NL = 3
EPS = 0.02
CL_W = 0.001
CL_T = 0.2
DECAY = 1e-4
E = 800000
B = 4096

SUB = 128
SUBS_PER_TILE = 408
E2 = SUBS_PER_TILE * 16 * SUB
EROWS = E2 // SUB
IBLK = 6
NBLK = SUBS_PER_TILE // IBLK
NT = IBLK // 3
FCH = 125
FITERS = (N // FCH) // 16

_MESH = plsc.VectorSubcoreMesh(core_axis_name="c", subcore_axis_name="s")


def _layer_body(with_noise, *refs):
    if with_noise:
        (x_r, ed_r, unit_r, acci_r, xo_r, acco_r,
         edata, rows, fb, acc_sp,
         semi0, semi1, semg0, semg1, semg2, sems0, sems1) = refs
    else:
        (x_r, ed_r, acci_r, xo_r, acco_r,
         edata, rows, fb, acc_sp,
         semi0, semi1, semg0, semg1, semg2, sems0, sems1) = refs
        unit_r = None

    c = lax.axis_index("c")
    s = lax.axis_index("s")
    semi = (semi0, semi1)
    semg = (semg0, semg1, semg2)

    seg_b = (rows.at[0].at[pl.ds(0, FCH)], rows.at[1].at[pl.ds(0, FCH)])
    aux_b = (rows.at[2].at[pl.ds(0, FCH)], fb.at[0])
    acm_b = (fb.at[1], fb.at[2])
    z16 = jnp.zeros((16,), jnp.float32)
    sub0 = s * SUBS_PER_TILE

    def _zbody(i, _):
        rows[0, i, 0:16] = z16
        rows[0, i, 16:32] = z16
        return 0

    lax.fori_loop(0, FCH, _zbody, 0)

    def _zcopy(k, _):
        pltpu.sync_copy(seg_b[0], acc_sp.at[pl.ds((s * FITERS + k) * FCH, FCH)])
        return 0

    lax.fori_loop(0, FITERS, _zcopy, 0)
    plsc.subcore_barrier()

    def _load_blk(bb, bslot):
        pltpu.async_copy(ed_r.at[pl.ds(sub0 + bb * IBLK, IBLK)],
                         edata.at[bslot], semi[bslot])

    def _drain_blk(bslot):
        pltpu.make_async_copy(ed_r.at[pl.ds(0, IBLK)], edata.at[bslot],
                              semi[bslot]).wait()

    def _issue_gather(bslot, jj, u):
        pltpu.async_copy(x_r.at[c].at[edata.at[bslot].at[jj].at[0]],
                         rows.at[u], semg[u])

    def _drain_gather(u):
        pltpu.make_async_copy(x_r.at[c].at[pl.ds(0, SUB)], rows.at[u],
                              semg[u]).wait()

    def _scat_sync(bslot, jj, u):
        pltpu.sync_copy(rows.at[u], acc_sp.at[edata.at[bslot].at[jj].at[1]],
                        add=True)

    def _scale(bslot, jj, u):
        def _sc16(q, _):
            wbits = edata[bslot, jj, 2, pl.ds(q * 16, 16)]
            w16 = lax.bitcast_convert_type(wbits, jnp.float32)
            for t in range(16):
                e16 = q * 16 + t
                wv = w16[t]
                rows[u, e16, 0:16] = rows[u, e16, 0:16] * wv
                rows[u, e16, 16:32] = rows[u, e16, 16:32] * wv
            return 0

        lax.fori_loop(0, SUB // 16, _sc16, 0)

    def _sub_step(bb, bslot, nbslot, t, u):
        jj = 3 * t + u
        un = (u + 1) % 3

        if u < 2:
            _issue_gather(bslot, jj + 1, un)
        else:
            @pl.when(t == 0)
            def _():
                _issue_gather(bslot, 3, un)

                @pl.when(bb + 1 < NBLK)
                def _():
                    _load_blk(bb + 1, nbslot)

            @pl.when(t == NT - 1)
            def _():
                @pl.when(bb + 1 < NBLK)
                def _():
                    _drain_blk(nbslot)
                    _issue_gather(nbslot, 0, un)

        _drain_gather(u)
        _scale(bslot, jj, u)
        _scat_sync(bslot, jj, u)

    _load_blk(0, 0)
    _drain_blk(0)
    _issue_gather(0, 0, 0)

    def _bpair(bp, _):
        def _triad(t, _2):
            for u in range(3):
                _sub_step(2 * bp, 0, 1, t, u)
            return 0

        lax.fori_loop(0, NT, _triad, 0)

        def _triad2(t, _2):
            for u in range(3):
                _sub_step(2 * bp + 1, 1, 0, t, u)
            return 0

        lax.fori_loop(0, NT, _triad2, 0)
        return 0

    lax.fori_loop(0, NBLK // 2, _bpair, 0)
    plsc.subcore_barrier()

    fseml = (semg0, semg1)
    fsems = (sems0, sems1)

    def _f_loads(k, sl):
        r0 = (s * FITERS + k) * FCH
        if with_noise:
            pltpu.async_copy(unit_r.at[c].at[pl.ds(r0, FCH)], aux_b[sl],
                             fseml[sl])
        pltpu.async_copy(acci_r.at[c].at[pl.ds(r0, FCH)], acm_b[sl], fseml[sl])

    def _f_drain_loads(sl):
        if with_noise:
            pltpu.make_async_copy(acci_r.at[c].at[pl.ds(0, FCH)], aux_b[sl],
                                  fseml[sl]).wait()
        pltpu.make_async_copy(acci_r.at[c].at[pl.ds(0, FCH)], acm_b[sl],
                              fseml[sl]).wait()

    def _f_stores(k, sl):
        r0 = (s * FITERS + k) * FCH
        pltpu.async_copy(seg_b[sl], xo_r.at[c].at[pl.ds(r0, FCH)], fsems[sl])
        pltpu.async_copy(acm_b[sl], acco_r.at[c].at[pl.ds(r0, FCH)], fsems[sl])

    def _f_drain_stores(sl):
        pltpu.make_async_copy(acci_r.at[c].at[pl.ds(0, FCH)], seg_b[sl],
                              fsems[sl]).wait()
        pltpu.make_async_copy(acci_r.at[c].at[pl.ds(0, FCH)], acm_b[sl],
                              fsems[sl]).wait()

    def _f_compute(sl):
        def _vb(i, _):
            s0 = rows[sl, i, 0:16]
            s1 = rows[sl, i, 16:32]
            if with_noise:
                if sl == 0:
                    a0 = rows[2, i, 0:16]
                    a1 = rows[2, i, 16:32]
                else:
                    a0 = fb[0, i, 0:16]
                    a1 = fb[0, i, 16:32]
                s0 = s0 + a0 * jnp.sign(s0)
                s1 = s1 + a1 * jnp.sign(s1)
                rows[sl, i, 0:16] = s0
                rows[sl, i, 16:32] = s1
            fb[1 + sl, i, 0:16] = fb[1 + sl, i, 0:16] + s0
            fb[1 + sl, i, 16:32] = fb[1 + sl, i, 16:32] + s1
            return 0

        lax.fori_loop(0, FCH, _vb, 0)

    def _f_chunk(k, sl, nsl):
        @pl.when(k >= 1)
        def _():
            _f_drain_stores(nsl)

        @pl.when(k + 1 < FITERS)
        def _():
            _f_loads(k + 1, nsl)

        pltpu.sync_copy(acc_sp.at[pl.ds((s * FITERS + k) * FCH, FCH)],
                        seg_b[sl])
        _f_drain_loads(sl)
        _f_compute(sl)
        _f_stores(k, sl)

    _f_loads(0, 0)

    def _fpair(fp, _):
        _f_chunk(2 * fp, 0, 1)
        _f_chunk(2 * fp + 1, 1, 0)
        return 0

    lax.fori_loop(0, FITERS // 2, _fpair, 0)
    _f_chunk(FITERS - 1, 0, 1)
    _f_drain_stores(0)


def _make_layer(with_noise):
    out_type = (
        jax.ShapeDtypeStruct((2, N, H), jnp.float32),
        jax.ShapeDtypeStruct((2, N, H), jnp.float32),
    )
    scratch = [
        pltpu.VMEM((2, IBLK, 3, SUB), jnp.int32),
        pltpu.VMEM((3, SUB, H), jnp.float32),
        pltpu.VMEM((3, FCH, H), jnp.float32),
        pltpu.VMEM_SHARED((N, H), jnp.float32),
    ] + [pltpu.SemaphoreType.DMA] * 7
    return pl.kernel(
        functools.partial(_layer_body, with_noise),
        out_type=out_type,
        mesh=_MESH,
        scratch_types=scratch,
        compiler_params=pltpu.CompilerParams(use_tc_tiling_on_sc=False),
    )


_layer_plain = _make_layer(False)
_layer_noise = _make_layer(True)


def _gather_body(m0, a1, a2, uemb, iemb, uix, pix, nix, pixn, nixn,
                 u_o, pi_o, ni_o, z1u_o, z2u_o, z1i_o, z2i_o,
                 u0_o, pi0_o, ni0_o,
                 idx_v, rows_v, idx1_v, rows64_v, sem):
    c = lax.axis_index("c")
    s = lax.axis_index("s")
    wid = s * 2 + c

    def split_gather(idx_hbm, table, out, load_idx):
        if load_idx:
            pltpu.sync_copy(idx_hbm.at[pl.ds(s * 256, 256)], idx_v)
        for j in range(2):
            pltpu.async_copy(table.at[c].at[idx_v.at[pl.ds(j * 128, 128)]],
                             rows_v.at[j], sem).wait()
            pltpu.sync_copy(rows_v.at[j],
                            out.at[c].at[pl.ds(s * 256 + j * 128, 128)])

    split_gather(uix, m0, u_o, True)
    split_gather(uix, a1, z1u_o, False)
    split_gather(uix, a2, z2u_o, False)
    split_gather(pixn, m0, pi_o, True)
    split_gather(pixn, a1, z1i_o, False)
    split_gather(pixn, a2, z2i_o, False)
    split_gather(nixn, m0, ni_o, True)

    def raw_gather(idx_hbm, table, out):
        pltpu.sync_copy(idx_hbm.at[pl.ds(wid * 128, 128)], idx1_v)
        pltpu.async_copy(table.at[idx1_v], rows64_v, sem).wait()
        pltpu.sync_copy(rows64_v, out.at[pl.ds(wid * 128, 128)])

    raw_gather(uix, uemb, u0_o)
    raw_gather(pix, iemb, pi0_o)
    raw_gather(nix, iemb, ni0_o)


_gather_k = pl.kernel(
    _gather_body,
    out_type=tuple(
        [jax.ShapeDtypeStruct((2, B, H), jnp.float32)] * 7
        + [jax.ShapeDtypeStruct((B, D), jnp.float32)] * 3
    ),
    mesh=_MESH,
    scratch_types=[
        pltpu.VMEM((256,), jnp.int32),
        pltpu.VMEM((2, 128, H), jnp.float32),
        pltpu.VMEM((128,), jnp.int32),
        pltpu.VMEM((128, D), jnp.float32),
        pltpu.SemaphoreType.DMA,
    ],
    compiler_params=pltpu.CompilerParams(use_tc_tiling_on_sc=False),
)


PREP_R = 1000
PREP_STEPS = N // PREP_R


def _prep_body(ue, ie, r0, r1, r2, r3, r4, r5,
               x0_o, o0, o1, o2, o3, o4, o5):
    i = pl.program_id(0)
    emb = jnp.where(i < (N_USERS // PREP_R), ue[...], ie[...])
    x0_o[0] = emb[:, :H]
    x0_o[1] = emb[:, H:]
    for r, o in ((r0, o0), (r1, o1), (r2, o2), (r3, o3), (r4, o4), (r5, o5)):
        x = r[...]
        nrm = jnp.sqrt(jnp.sum(x * x, axis=1, keepdims=True))
        un = x / (nrm + 1e-12) * EPS
        o[0] = un[:, :H]
        o[1] = un[:, H:]


def _prep(user_emb, item_emb, rs):
    half = N_USERS // PREP_R
    r_spec = pl.BlockSpec((PREP_R, D), lambda i: (i, 0))
    o_spec = pl.BlockSpec((2, PREP_R, H), lambda i: (0, i, 0))
    return pl.pallas_call(
        _prep_body,
        grid=(PREP_STEPS,),
        in_specs=[
            pl.BlockSpec((PREP_R, D), lambda i: (jnp.minimum(i, half - 1), 0)),
            pl.BlockSpec((PREP_R, D), lambda i: (jnp.maximum(i - half, 0), 0)),
        ] + [r_spec] * 6,
        out_specs=[o_spec] * 7,
        out_shape=[jax.ShapeDtypeStruct((2, N, H), jnp.float32)] * 7,
    )(user_emb, item_emb, *rs)


RB = 512
RSTEPS = B // RB


def _nce_sum(z1b, z2f, z2b):
    z1lo = z1b[0] * 0.25
    z1hi = z1b[1] * 0.25
    n1 = jnp.sqrt(jnp.sum(z1lo * z1lo, axis=1, keepdims=True)
                  + jnp.sum(z1hi * z1hi, axis=1, keepdims=True))
    i1 = 1.0 / (n1 + 1e-12)
    z1lo = z1lo * i1
    z1hi = z1hi * i1
    z2lo = z2f[0] * 0.25
    z2hi = z2f[1] * 0.25
    n2 = jnp.sqrt(jnp.sum(z2lo * z2lo, axis=1, keepdims=True)
                  + jnp.sum(z2hi * z2hi, axis=1, keepdims=True))
    i2 = 1.0 / (n2 + 1e-12)
    z2lo = z2lo * i2
    z2hi = z2hi * i2
    nt = (((1,), (1,)), ((), ()))
    sim = (lax.dot_general(z1lo, z2lo, nt, preferred_element_type=jnp.float32)
           + lax.dot_general(z1hi, z2hi, nt, preferred_element_type=jnp.float32))
    sim = sim * (1.0 / CL_T)
    m = jnp.max(sim, axis=1, keepdims=True)
    logz = jnp.log(jnp.sum(jnp.exp(sim - m), axis=1, keepdims=True)) + m
    z2blo = z2b[0] * 0.25
    z2bhi = z2b[1] * 0.25
    nb = jnp.sqrt(jnp.sum(z2blo * z2blo, axis=1, keepdims=True)
                  + jnp.sum(z2bhi * z2bhi, axis=1, keepdims=True))
    ib = 1.0 / (nb + 1e-12)
    diag = (jnp.sum(z1lo * (z2blo * ib), axis=1, keepdims=True)
            + jnp.sum(z1hi * (z2bhi * ib), axis=1, keepdims=True)) * (1.0 / CL_T)
    return jnp.sum(logz - diag)


def _loss_body(z1u, z2uf, z2ub, z1i, z2if, z2ib, u, pi, ni, u0, pi0, ni0,
               loss_o, bpr_o, acc):
    i = pl.program_id(0)

    @pl.when(i == 0)
    def _():
        ps = jnp.sum(u[0] * pi[0] + u[1] * pi[1], axis=1) * (1.0 / 16.0)
        ns = jnp.sum(u[0] * ni[0] + u[1] * ni[1], axis=1) * (1.0 / 16.0)
        d = ps - ns
        sp = jnp.maximum(-d, 0.0) + jnp.log(1.0 + jnp.exp(-jnp.abs(d)))
        acc[0] = jnp.mean(sp)
        acc[1] = (jnp.sum(u0[...] * u0[...]) + jnp.sum(pi0[...] * pi0[...])
                  + jnp.sum(ni0[...] * ni0[...])) * (1.0 / B)
        acc[2] = 0.0
        acc[3] = 0.0

    acc[2] += _nce_sum(z1u, z2uf, z2ub)
    acc[3] += _nce_sum(z1i, z2if, z2ib)

    @pl.when(i == RSTEPS - 1)
    def _():
        cl = acc[2] * (1.0 / B) + acc[3] * (1.0 / B)
        loss_o[...] = jnp.broadcast_to(acc[0] + DECAY * acc[1] + CL_W * cl,
                                       (1, 1))
        bpr_o[...] = jnp.broadcast_to(acc[0], (1, 1))


def _loss(z1u, z2u, z1i, z2i, u, pi, ni, u0, pi0, ni0):
    full = pl.BlockSpec((2, B, H), lambda i: (0, 0, 0))
    blk = pl.BlockSpec((2, RB, H), lambda i: (0, i, 0))
    raw = pl.BlockSpec((B, D), lambda i: (0, 0))
    return pl.pallas_call(
        _loss_body,
        grid=(RSTEPS,),
        in_specs=[blk, full, blk, blk, full, blk,
                  full, full, full, raw, raw, raw],
        out_specs=[pl.BlockSpec((1, 1), lambda i: (0, 0))] * 2,
        out_shape=[jax.ShapeDtypeStruct((1, 1), jnp.float32)] * 2,
        scratch_shapes=[pltpu.SMEM((4,), jnp.float32)],
    )(z1u, z2u, z2u, z1i, z2i, z2i, u, pi, ni, u0, pi0, ni0)


def kernel(users, pos_items, neg_items, edge_index, edge_weight,
           user_emb, item_emb):
    users = users.astype(jnp.int32)
    pos_items = pos_items.astype(jnp.int32)
    neg_items = neg_items.astype(jnp.int32)
    pad = E2 - E
    pad_idx = (jnp.arange(pad, dtype=jnp.int32) * 37) % N
    src = jnp.concatenate(
        [edge_index[0].astype(jnp.int32), pad_idx]).reshape(EROWS, 1, SUB)
    dst = jnp.concatenate(
        [edge_index[1].astype(jnp.int32), pad_idx]).reshape(EROWS, 1, SUB)
    wbits = lax.bitcast_convert_type(
        jnp.concatenate([edge_weight, jnp.zeros((pad,), jnp.float32)]),
        jnp.int32).reshape(EROWS, 1, SUB)
    ed = jnp.concatenate([src, dst, wbits], axis=1)

    rs = [jax.random.uniform(jax.random.fold_in(jax.random.key(k), l),
                             (N, D), dtype=jnp.float32)
          for k in (1, 2) for l in range(NL)]
    x0, u1a, u1b, u1c, u2a, u2b, u2c = _prep(user_emb, item_emb, rs)

    x, acc = x0, x0
    for _ in range(NL):
        x, acc = _layer_plain(x, ed, acc)
    m0 = acc

    x, acc = x0, x0
    for un in (u1a, u1b, u1c):
        x, acc = _layer_noise(x, ed, un, acc)
    a1 = acc
    x, acc = x0, x0
    for un in (u2a, u2b, u2c):
        x, acc = _layer_noise(x, ed, un, acc)
    a2 = acc

    uix = users
    pix = pos_items
    nix = neg_items
    pixn = pix + N_USERS
    nixn = nix + N_USERS

    (u, pi, ni, z1u, z2u, z1i, z2i, u0, pi0, ni0) = _gather_k(
        m0, a1, a2, user_emb, item_emb, uix, pix, nix, pixn, nixn)

    loss_a, bpr_a = _loss(z1u, z2u, z1i, z2i, u, pi, ni, u0, pi0, ni0)
    return (loss_a[0, 0], bpr_a[0, 0])
